# baseline (speedup 1.0000x reference)
import functools

import jax
import jax.numpy as jnp
from jax import lax
from jax.experimental import pallas as pl
from jax.experimental.pallas import tpu as pltpu
from jax.experimental.pallas import tpu_sc as plsc

DIM = 64
NC = 2
NS = 16
NW = NC * NS
CHUNK = 128


@functools.cache
def _make_gather(B):
    b_per_w = B // NW
    n_chunks = b_per_w // CHUNK
    mesh = plsc.VectorSubcoreMesh(core_axis_name="c", subcore_axis_name="s")

    @functools.partial(
        pl.kernel,
        mesh=mesh,
        compiler_params=pltpu.CompilerParams(use_tc_tiling_on_sc=False),
        out_type=jax.ShapeDtypeStruct((B, DIM), jnp.float32),
        scratch_types=[
            pltpu.VMEM((n_chunks, CHUNK), jnp.int32),
            pltpu.VMEM((CHUNK, DIM), jnp.float32),
            pltpu.SemaphoreType.DMA,
        ],
    )
    def k(idx_hbm, table_hbm, out_hbm, idx_v, rows_v, sem):
        wid = lax.axis_index("s") * NC + lax.axis_index("c")
        base = wid * b_per_w
        pltpu.sync_copy(idx_hbm.at[wid], idx_v)

        def body(j, carry):
            pltpu.async_copy(table_hbm.at[idx_v.at[j]], rows_v, sem).wait()
            pltpu.sync_copy(rows_v, out_hbm.at[pl.ds(base + j * CHUNK, CHUNK)])
            return carry

        lax.fori_loop(0, n_chunks, body, 0)

    return k


def kernel(xs, weight):
    B = xs.shape[0] * xs.shape[1]
    idx = xs.astype(jnp.int32).reshape(NW, -1, CHUNK)
    out = _make_gather(B)(idx, weight)
    return out.reshape(xs.shape[0], xs.shape[1], DIM)


# speedup vs baseline: 1.0357x; 1.0357x over previous
import functools

import jax
import jax.numpy as jnp
from jax import lax
from jax.experimental import pallas as pl
from jax.experimental.pallas import tpu as pltpu
from jax.experimental.pallas import tpu_sc as plsc

DIM = 64
NC = 2
NS = 16
NW = NC * NS
CHUNK = 640


@functools.cache
def _make_gather(B):
    b_per_w = B // NW
    n_chunks = b_per_w // CHUNK
    mesh = plsc.VectorSubcoreMesh(core_axis_name="c", subcore_axis_name="s")

    @functools.partial(
        pl.kernel,
        mesh=mesh,
        compiler_params=pltpu.CompilerParams(use_tc_tiling_on_sc=False),
        out_type=jax.ShapeDtypeStruct((B, DIM), jnp.float32),
        scratch_types=[
            pltpu.VMEM((n_chunks, CHUNK), jnp.int32),
            pltpu.VMEM((CHUNK, DIM), jnp.float32),
            pltpu.SemaphoreType.DMA,
        ],
    )
    def k(idx_hbm, table_hbm, out_hbm, idx_v, rows_v, sem):
        wid = lax.axis_index("s") * NC + lax.axis_index("c")
        base = wid * b_per_w
        pltpu.sync_copy(idx_hbm.at[wid], idx_v)

        def body(j, carry):
            pltpu.async_copy(table_hbm.at[idx_v.at[j]], rows_v, sem).wait()
            pltpu.sync_copy(rows_v, out_hbm.at[pl.ds(base + j * CHUNK, CHUNK)])
            return carry

        lax.fori_loop(0, n_chunks, body, 0)

    return k


def kernel(xs, weight):
    B = xs.shape[0] * xs.shape[1]
    idx = xs.astype(jnp.int32).reshape(NW, -1, CHUNK)
    out = _make_gather(B)(idx, weight)
    return out.reshape(xs.shape[0], xs.shape[1], DIM)


# speedup vs baseline: 1.0372x; 1.0015x over previous
import functools

import jax
import jax.numpy as jnp
from jax import lax
from jax.experimental import pallas as pl
from jax.experimental.pallas import tpu as pltpu
from jax.experimental.pallas import tpu_sc as plsc

DIM = 64
NC = 2
NS = 16
NW = NC * NS
SEQ_CHUNK = 16
N_CHUNKS = 8


@functools.cache
def _make_gather(B4, S):
    seq_per_w = B4 // NW
    chunk = SEQ_CHUNK * S
    assert seq_per_w == SEQ_CHUNK * N_CHUNKS
    mesh = plsc.VectorSubcoreMesh(core_axis_name="c", subcore_axis_name="s")

    @functools.partial(
        pl.kernel,
        mesh=mesh,
        compiler_params=pltpu.CompilerParams(use_tc_tiling_on_sc=False),
        out_type=jax.ShapeDtypeStruct((B4, S, DIM), jnp.float32),
        scratch_types=[
            pltpu.VMEM((N_CHUNKS, chunk), jnp.int32),
            pltpu.VMEM((chunk, DIM), jnp.float32),
            pltpu.SemaphoreType.DMA,
            pltpu.SemaphoreType.DMA,
        ],
    )
    def k(idx_hbm, table_hbm, out_hbm, idx_v, rows_v, sem, wsem):
        wid = lax.axis_index("s") * NC + lax.axis_index("c")
        base = wid * seq_per_w
        pltpu.sync_copy(idx_hbm.at[wid], idx_v)

        def body(j, carry):
            pltpu.async_copy(table_hbm.at[idx_v.at[j]], rows_v, sem).wait()
            b0 = base + j * SEQ_CHUNK
            cps = [
                pltpu.async_copy(rows_v.at[pl.ds(i * S, S)], out_hbm.at[b0 + i], wsem)
                for i in range(SEQ_CHUNK)
            ]
            for cp in cps:
                cp.wait()
            return carry

        lax.fori_loop(0, N_CHUNKS, body, 0)

    return k


def kernel(xs, weight):
    idx = xs.astype(jnp.int32).reshape(NW, N_CHUNKS, SEQ_CHUNK * xs.shape[1])
    return _make_gather(xs.shape[0], xs.shape[1])(idx, weight)
